# Initial kernel scaffold; baseline (speedup 1.0000x reference)
import jax
import jax.numpy as jnp
from jax.experimental import pallas as pl


def kernel(user_indices, item_indices, embed_user_GMF, embed_item_GMF, embed_user_MLP, embed_item_MLP, W1, b1, W2, b2, Wp, bp):
    raise NotImplementedError("write your pallas kernel here")


# speedup vs baseline: 1.7648x; 1.7648x over previous
import functools

import jax
import jax.numpy as jnp
from jax import lax
from jax.experimental import pallas as pl
from jax.experimental.pallas import tpu as pltpu
from jax.experimental.pallas import tpu_sc as plsc

B = 16384
D = 64
W = 2 * D
NC, NS = 2, 16
NW = NC * NS
ROWS_W = B // NW
CHUNK = 128
NCHUNK = ROWS_W // CHUNK

_sc_mesh = plsc.VectorSubcoreMesh(core_axis_name="c", subcore_axis_name="s")


@functools.partial(
    pl.kernel,
    out_type=(
        jax.ShapeDtypeStruct((B, W), jnp.float32),
        jax.ShapeDtypeStruct((B, W), jnp.float32),
    ),
    mesh=_sc_mesh,
    scratch_types=(
        pltpu.VMEM((NCHUNK, CHUNK), jnp.int32),
        pltpu.VMEM((NCHUNK, CHUNK), jnp.int32),
        pltpu.VMEM((CHUNK, W), jnp.float32),
        pltpu.VMEM((CHUNK, W), jnp.float32),
        pltpu.SemaphoreType.DMA,
        pltpu.SemaphoreType.DMA,
    ),
)
def _sc_gather(uidx_hbm, iidx_hbm, t_u, t_i,
               out_u, out_i,
               idx_u, idx_i, buf0, buf1, sem0, sem1):
    wid = lax.axis_index("s") * NC + lax.axis_index("c")
    base = wid * ROWS_W
    for j in range(NCHUNK):
        pltpu.sync_copy(uidx_hbm.at[pl.ds(base + j * CHUNK, CHUNK)], idx_u.at[j])
        pltpu.sync_copy(iidx_hbm.at[pl.ds(base + j * CHUNK, CHUNK)], idx_i.at[j])

    seq = []
    for tab, idx, out in ((t_u, idx_u, out_u), (t_i, idx_i, out_i)):
        for j in range(NCHUNK):
            seq.append((tab, idx.at[j], out, j))

    bufs = (buf0, buf1)
    sems = (sem0, sem1)
    copies = [None, None]
    for k, (tab, idxv, out, j) in enumerate(seq):
        copies[k % 2] = pltpu.async_copy(tab.at[idxv], bufs[k % 2], sems[k % 2])
        if k > 0:
            prev = seq[k - 1]
            copies[(k - 1) % 2].wait()
            pltpu.sync_copy(bufs[(k - 1) % 2],
                            prev[2].at[pl.ds(base + prev[3] * CHUNK, CHUNK)])
    k = len(seq) - 1
    copies[k % 2].wait()
    pltpu.sync_copy(bufs[k % 2], seq[k][2].at[pl.ds(base + seq[k][3] * CHUNK, CHUNK)])


def _dense_body(u, i, w1u, w1i, b1, w2, b2, wpg, wph, bp, out):
    uv = u[:]
    iv = i[:]
    g = uv[:, :D] * iv[:, :D]
    h1 = jnp.maximum(
        jnp.dot(uv[:, D:], w1u[:], preferred_element_type=jnp.float32)
        + jnp.dot(iv[:, D:], w1i[:], preferred_element_type=jnp.float32)
        + b1[:], 0.0)
    h2 = jnp.maximum(
        jnp.dot(h1, w2[:], preferred_element_type=jnp.float32) + b2[:], 0.0)
    logit = (jnp.sum(g * wpg[:], axis=1) + jnp.sum(h2 * wph[:], axis=1)
             + bp[0, 0])
    out[:] = jax.nn.sigmoid(logit)


_R = 2048


def kernel(user_indices, item_indices, embed_user_GMF, embed_item_GMF,
           embed_user_MLP, embed_item_MLP, W1, b1, W2, b2, Wp, bp):
    cat_u = jnp.concatenate([embed_user_GMF, embed_user_MLP], axis=1)
    cat_i = jnp.concatenate([embed_item_GMF, embed_item_MLP], axis=1)
    u_rows, i_rows = _sc_gather(user_indices, item_indices, cat_u, cat_i)

    w1u = W1[:D]
    w1i = W1[D:]
    wpg = Wp[:D, 0].reshape(1, D)
    wph = Wp[D:, 0].reshape(1, 32)
    b1r = b1.reshape(1, 64)
    b2r = b2.reshape(1, 32)
    bpr = bp.reshape(1, 1)

    full = lambda shape: pl.BlockSpec(shape, lambda i: (0, 0))
    out = pl.pallas_call(
        _dense_body,
        grid=(B // _R,),
        in_specs=[
            pl.BlockSpec((_R, W), lambda i: (i, 0)),
            pl.BlockSpec((_R, W), lambda i: (i, 0)),
            full((D, 64)), full((D, 64)), full((1, 64)),
            full((64, 32)), full((1, 32)),
            full((1, D)), full((1, 32)), full((1, 1)),
        ],
        out_specs=pl.BlockSpec((_R,), lambda i: (i,)),
        out_shape=jax.ShapeDtypeStruct((B,), jnp.float32),
    )(u_rows, i_rows, w1u, w1i, b1r, W2, b2r, wpg, wph, bpr)
    return out
